# baseline (speedup 1.0000x reference)
import functools

import jax
import jax.numpy as jnp
from jax import lax
from jax.experimental import pallas as pl
from jax.experimental.pallas import tpu as pltpu
from jax.experimental.pallas import tpu_sc as plsc

_VOCAB = 32002
_HIDDEN = 2048
_BATCH = 4
_SEQ = 4096
_NTOK = _BATCH * _SEQ
_NW = 32
_PER_W = _NTOK // _NW
_CHUNK = 8
_NCHUNK = _PER_W // _CHUNK
_NPAIR = _NCHUNK // 2

_mesh = plsc.VectorSubcoreMesh(core_axis_name="c", subcore_axis_name="s")


@functools.partial(
    pl.kernel,
    out_type=jax.ShapeDtypeStruct((_NTOK, _HIDDEN), jnp.float32),
    mesh=_mesh,
    scratch_types=(
        [pltpu.VMEM((_NCHUNK, _CHUNK), jnp.int32)]
        + [pltpu.VMEM((_CHUNK, _HIDDEN), jnp.float32)] * 2
        + [pltpu.VMEM_SHARED((16, 2, 2 * _CHUNK, _HIDDEN), jnp.float32)]
        + [pltpu.SemaphoreType.DMA] * 6
    ),
)
def _embed_lookup(table_hbm, idx_hbm, out_hbm, idx_v, buf0, buf1, shared,
                  g0, g1, x0, x1, o0, o1):
    sid = lax.axis_index("s")
    wid = sid * 2 + lax.axis_index("c")
    base = wid * _PER_W
    pltpu.sync_copy(idx_hbm.at[wid], idx_v)

    bufs = (buf0, buf1)
    gsems = (g0, g1)
    xsems = (x0, x1)
    osems = (o0, o1)

    def gather_start(j, g):
        pltpu.async_copy(table_hbm.at[idx_v.at[j]], bufs[g], gsems[g])

    def gather_wait(g):
        pltpu.make_async_copy(
            table_hbm.at[idx_v.at[0]], bufs[g], gsems[g]).wait()

    def stage(g, o, half):
        pltpu.async_copy(
            bufs[g],
            shared.at[sid, o, pl.ds(half * _CHUNK, _CHUNK)], xsems[g]).wait()

    def out_start(p, o):
        pltpu.async_copy(
            shared.at[sid, o],
            out_hbm.at[pl.ds(base + p * 2 * _CHUNK, 2 * _CHUNK)], osems[o])

    def out_wait(o):
        pltpu.make_async_copy(
            shared.at[sid, o],
            out_hbm.at[pl.ds(base, 2 * _CHUNK)], osems[o]).wait()

    def pair_body(p, o, skip_out_wait=False, prefetch=True):
        if not skip_out_wait:
            out_wait(o)
        for g in range(2):
            gather_wait(g)
            stage(g, o, g)
            if prefetch:
                gather_start(2 * p + 2 + g, g)
        out_start(p, o)

    gather_start(0, 0)
    gather_start(1, 1)
    pair_body(0, 0, skip_out_wait=True)
    pair_body(1, 1, skip_out_wait=True)

    def step(k, carry):
        for o in range(2):
            pair_body(2 * k + o, o)
        return carry

    lax.fori_loop(1, _NPAIR // 2 - 1, step, 0)

    pair_body(_NPAIR - 2, 0)
    pair_body(_NPAIR - 1, 1, prefetch=False)
    out_wait(0)
    out_wait(1)


def kernel(input_ids, embedding_table):
    ids = input_ids.reshape(_NW, _NCHUNK, _CHUNK)
    out = _embed_lookup(embedding_table, ids)
    return out.reshape(_BATCH, _SEQ, _HIDDEN)
